# baseline (speedup 1.0000x reference)
import functools

import jax
import jax.numpy as jnp
from jax import lax
from jax.experimental import pallas as pl
from jax.experimental.pallas import tpu as pltpu
from jax.experimental.pallas import tpu_sc as plsc

NC = 2
NS = 16
NW = NC * NS
L = 16


def _sc_fact_bank(batch, dim, num_facts, chunk, rg):
    rows_per_w = batch // NW
    n_chunks = num_facts // chunk
    n_vec = chunk // L
    n_groups = rows_per_w // rg
    assert n_groups == 2
    mesh = plsc.VectorSubcoreMesh(core_axis_name="c", subcore_axis_name="s")

    @functools.partial(
        pl.kernel,
        mesh=mesh,
        out_type=jax.ShapeDtypeStruct((batch * num_facts,), jnp.float32),
        compiler_params=pltpu.CompilerParams(needs_layout_passes=False),
        scratch_types=[
            pltpu.VMEM((rows_per_w * dim,), jnp.float32),
            pltpu.VMEM((chunk,), jnp.int32),
            pltpu.VMEM((chunk,), jnp.int32),
            pltpu.VMEM((chunk,), jnp.float32),
            pltpu.VMEM((chunk,), jnp.float32),
            pltpu.VMEM((rg * chunk,), jnp.float32),
            pltpu.VMEM((rg * chunk,), jnp.float32),
            pltpu.SemaphoreType.DMA,
            pltpu.SemaphoreType.DMA,
        ],
    )
    def k(x_hbm, th_hbm, lk_hbm, ii_hbm, jj_hbm, out_hbm,
          xflat, iv, jv, nkv, av, ob0, ob1, sem0, sem1):
        wid = lax.axis_index("s") * NC + lax.axis_index("c")
        base = wid * rows_per_w
        pltpu.sync_copy(x_hbm.at[pl.ds(base * dim, rows_per_w * dim)], xflat)

        def drain(ob, sem):
            for q in range(rg):
                pltpu.make_async_copy(
                    ob.at[pl.ds(q * chunk, chunk)],
                    out_hbm.at[pl.ds(q * chunk, chunk)], sem).wait()

        def compute_group(ob, row0):
            @plsc.parallel_loop(0, n_vec, unroll=2)
            def vec_body(v):
                s = pl.ds(v * L, L)
                nk = nkv[s]
                a = av[s]
                fi = iv[s] + (row0 * dim)
                fj = jv[s] + (row0 * dim)
                for q in range(rg):
                    xi = plsc.load_gather(xflat, [fi])
                    xj = plsc.load_gather(xflat, [fj])
                    e = jnp.exp(nk * (xi - xj) + a)
                    ob[pl.ds(q * chunk + v * L, L)] = 1.0 / (1.0 + e)
                    if q != rg - 1:
                        fi = fi + dim
                        fj = fj + dim

        def fire(ob, sem, row0, off):
            for q in range(rg):
                dst = pl.ds((base + row0 + q) * num_facts + off, chunk)
                pltpu.async_copy(
                    ob.at[pl.ds(q * chunk, chunk)], out_hbm.at[dst], sem)

        def chunk_body(c, _):
            off = pl.multiple_of(c * chunk, 256)
            pltpu.sync_copy(ii_hbm.at[pl.ds(off, chunk)], iv)
            pltpu.sync_copy(jj_hbm.at[pl.ds(off, chunk)], jv)
            pltpu.sync_copy(th_hbm.at[pl.ds(off, chunk)], nkv)
            pltpu.sync_copy(lk_hbm.at[pl.ds(off, chunk)], av)

            @plsc.parallel_loop(0, n_vec)
            def pre(v):
                s = pl.ds(v * L, L)
                kap = jnp.clip(jnp.exp(av[s]), 0.5, 50.0)
                av[s] = kap * nkv[s]
                nkv[s] = -kap

            @pl.when(c > 0)
            def _drain0():
                drain(ob0, sem0)

            compute_group(ob0, 0)
            fire(ob0, sem0, 0, off)

            @pl.when(c > 0)
            def _drain1():
                drain(ob1, sem1)

            compute_group(ob1, rg)
            fire(ob1, sem1, rg, off)
            return _

        lax.fori_loop(0, n_chunks, chunk_body, None)
        drain(ob0, sem0)
        drain(ob1, sem1)

    return k


def kernel(x, th, log_kappa, i_idx, j_idx):
    batch, dim = x.shape
    num_facts = i_idx.shape[0]
    chunk = 2176
    k = _sc_fact_bank(batch, dim, num_facts, chunk, rg=16)
    out = k(x.reshape(-1), th, log_kappa, i_idx, j_idx)
    return out.reshape(batch, num_facts)


# speedup vs baseline: 1.9655x; 1.9655x over previous
import functools

import jax
import jax.numpy as jnp
from jax import lax
from jax.experimental import pallas as pl
from jax.experimental.pallas import tpu as pltpu
from jax.experimental.pallas import tpu_sc as plsc

NC = 2
NS = 16
NW = NC * NS
L = 16


def _sc_fact_bank(batch, dim, num_facts, chunk, rg):
    rows_per_w = batch // NW
    n_chunks = num_facts // chunk
    n_vec = chunk // L
    n_groups = rows_per_w // rg
    mesh = plsc.VectorSubcoreMesh(core_axis_name="c", subcore_axis_name="s")

    @functools.partial(
        pl.kernel,
        mesh=mesh,
        out_type=jax.ShapeDtypeStruct((batch * num_facts,), jnp.float32),
        compiler_params=pltpu.CompilerParams(needs_layout_passes=False),
        scratch_types=[
            pltpu.VMEM((rows_per_w * dim,), jnp.float32),
            pltpu.VMEM((chunk,), jnp.int32),
            pltpu.VMEM((chunk,), jnp.int32),
            pltpu.VMEM((chunk,), jnp.float32),
            pltpu.VMEM((chunk,), jnp.float32),
            pltpu.VMEM((rg * chunk,), jnp.float32),
            pltpu.VMEM((rg * chunk,), jnp.float32),
            pltpu.SemaphoreType.DMA,
            pltpu.SemaphoreType.DMA,
        ],
    )
    def k(x_hbm, th_hbm, lk_hbm, ii_hbm, jj_hbm, out_hbm,
          xflat, iv, jv, nkv, av, ob0, ob1, sem0, sem1):
        wid = lax.axis_index("s") * NC + lax.axis_index("c")
        base = wid * rows_per_w
        pltpu.sync_copy(x_hbm.at[pl.ds(base * dim, rows_per_w * dim)], xflat)

        def drain(ob, sem):
            for q in range(rg):
                pltpu.make_async_copy(
                    ob.at[pl.ds(q * chunk, chunk)],
                    out_hbm.at[pl.ds(q * chunk, chunk)], sem).wait()

        def compute_group(ob, row0):
            @plsc.parallel_loop(0, n_vec)
            def vec_body(v):
                s = pl.ds(v * L, L)
                nk = nkv[s]
                a = av[s]
                fi = iv[s] + (row0 * dim)
                fj = jv[s] + (row0 * dim)
                for q in range(rg):
                    xi = plsc.load_gather(xflat, [fi])
                    xj = plsc.load_gather(xflat, [fj])
                    e = jnp.exp(nk * (xi - xj) + a)
                    ob[pl.ds(q * chunk + v * L, L)] = 1.0 / (1.0 + e)
                    if q != rg - 1:
                        fi = fi + dim
                        fj = fj + dim

        def fire(ob, sem, row0, off):
            for q in range(rg):
                dst = pl.ds((base + row0 + q) * num_facts + off, chunk)
                pltpu.async_copy(
                    ob.at[pl.ds(q * chunk, chunk)], out_hbm.at[dst], sem)

        def chunk_body(c, _):
            off = pl.multiple_of(c * chunk, 256)
            pltpu.sync_copy(ii_hbm.at[pl.ds(off, chunk)], iv)
            pltpu.sync_copy(jj_hbm.at[pl.ds(off, chunk)], jv)
            pltpu.sync_copy(th_hbm.at[pl.ds(off, chunk)], nkv)
            pltpu.sync_copy(lk_hbm.at[pl.ds(off, chunk)], av)

            @plsc.parallel_loop(0, n_vec)
            def pre(v):
                s = pl.ds(v * L, L)
                kap = jnp.clip(jnp.exp(av[s]), 0.5, 50.0)
                av[s] = kap * nkv[s]
                nkv[s] = -kap

            for g in range(n_groups):
                ob, sem = (ob0, sem0) if g % 2 == 0 else (ob1, sem1)
                if g >= 2:
                    drain(ob, sem)
                else:
                    @pl.when(c > 0)
                    def _d(ob=ob, sem=sem):
                        drain(ob, sem)

                compute_group(ob, g * rg)
                fire(ob, sem, g * rg, off)
            return _

        lax.fori_loop(0, n_chunks, chunk_body, None)
        drain(ob0, sem0)
        drain(ob1, sem1)

    return k


def kernel(x, th, log_kappa, i_idx, j_idx):
    batch, dim = x.shape
    num_facts = i_idx.shape[0]
    chunk = 2176
    k = _sc_fact_bank(batch, dim, num_facts, chunk, rg=8)
    out = k(x.reshape(-1), th, log_kappa, i_idx, j_idx)
    return out.reshape(batch, num_facts)


# speedup vs baseline: 11.9595x; 6.0847x over previous
import functools

import jax
import jax.numpy as jnp
from jax import lax
from jax.experimental import pallas as pl
from jax.experimental.pallas import tpu as pltpu
from jax.experimental.pallas import tpu_sc as plsc

NC = 2
NS = 16
NW = NC * NS
L = 16


def _sc_fact_bank(batch, dim, num_facts, chunk, rg):
    rows_per_w = batch // NW
    n_chunks = num_facts // chunk
    n_vec = chunk // L
    n_groups = max(rows_per_w // rg, 1)
    rg = min(rg, rows_per_w)
    mesh = plsc.VectorSubcoreMesh(core_axis_name="c", subcore_axis_name="s")

    @functools.partial(
        pl.kernel,
        mesh=mesh,
        out_type=jax.ShapeDtypeStruct((batch * num_facts,), jnp.float32),
        compiler_params=pltpu.CompilerParams(needs_layout_passes=False),
        scratch_types=[
            pltpu.VMEM((rows_per_w * dim,), jnp.float32),
            pltpu.VMEM((chunk,), jnp.int32),
            pltpu.VMEM((chunk,), jnp.int32),
            pltpu.VMEM((chunk,), jnp.float32),
            pltpu.VMEM((chunk,), jnp.float32),
            pltpu.VMEM((rg * chunk,), jnp.float32),
            pltpu.VMEM((rg * chunk,), jnp.float32),
            pltpu.SemaphoreType.DMA,
            pltpu.SemaphoreType.DMA,
        ],
    )
    def k(x_hbm, th_hbm, lk_hbm, ii_hbm, jj_hbm, out_hbm,
          xflat, iv, jv, nkv, av, ob0, ob1, sem0, sem1):
        wid = lax.axis_index("s") * NC + lax.axis_index("c")
        base = wid * rows_per_w
        pltpu.sync_copy(x_hbm.at[pl.ds(base * dim, rows_per_w * dim)], xflat)

        def drain(ob, sem):
            for q in range(rg):
                pltpu.make_async_copy(
                    ob.at[pl.ds(q * chunk, chunk)],
                    out_hbm.at[pl.ds(q * chunk, chunk)], sem).wait()

        def compute_group(ob, row0):
            @plsc.parallel_loop(0, n_vec)
            def vec_body(v):
                s = pl.ds(v * L, L)
                nk = nkv[s]
                a = av[s]
                fi = iv[s] + (row0 * dim)
                fj = jv[s] + (row0 * dim)
                for q in range(rg):
                    xi = plsc.load_gather(xflat, [fi])
                    xj = plsc.load_gather(xflat, [fj])
                    e = jnp.exp(nk * (xi - xj) + a)
                    ob[pl.ds(q * chunk + v * L, L)] = 1.0 / (1.0 + e)
                    if q != rg - 1:
                        fi = fi + dim
                        fj = fj + dim

        def fire(ob, sem, row0, off):
            for q in range(rg):
                dst = pl.ds((base + row0 + q) * num_facts + off, chunk)
                pltpu.async_copy(
                    ob.at[pl.ds(q * chunk, chunk)], out_hbm.at[dst], sem)

        def chunk_body(c, _):
            off = pl.multiple_of(c * chunk, 256)
            pltpu.sync_copy(ii_hbm.at[pl.ds(off, chunk)], iv)
            pltpu.sync_copy(jj_hbm.at[pl.ds(off, chunk)], jv)
            pltpu.sync_copy(th_hbm.at[pl.ds(off, chunk)], nkv)
            pltpu.sync_copy(lk_hbm.at[pl.ds(off, chunk)], av)

            @plsc.parallel_loop(0, n_vec)
            def pre(v):
                s = pl.ds(v * L, L)
                kap = jnp.clip(jnp.exp(av[s]), 0.5, 50.0)
                av[s] = kap * nkv[s]
                nkv[s] = -kap

            for g in range(n_groups):
                ob, sem = (ob0, sem0) if g % 2 == 0 else (ob1, sem1)
                if g >= 2:
                    drain(ob, sem)
                else:
                    @pl.when(c > 0)
                    def _d(ob=ob, sem=sem):
                        drain(ob, sem)

                compute_group(ob, g * rg)
                fire(ob, sem, g * rg, off)
            return _

        lax.fori_loop(0, n_chunks, chunk_body, None)
        drain(ob0, sem0)
        if n_groups > 1:
            drain(ob1, sem1)

    return k


def _sc_part(x, th, log_kappa, i_idx, j_idx):
    batch, dim = x.shape
    num_facts = i_idx.shape[0]
    chunk = 2176
    k = _sc_fact_bank(batch, dim, num_facts, chunk, rg=16)
    out = k(x.reshape(-1), th, log_kappa, i_idx, j_idx)
    return out.reshape(batch, num_facts)


def _tc_kernel_body(ii_ref, jj_ref, th_ref, lk_ref, x_ref, o_ref):
    fb = ii_ref.shape[-1]
    dim = x_ref.shape[-1]
    ii = ii_ref[0]
    jj = jj_ref[0]
    row = lax.broadcasted_iota(jnp.int32, (dim, fb), 0)
    sel = jnp.where(row == ii, 1.0, 0.0) - jnp.where(row == jj, 1.0, 0.0)
    z = jnp.dot(x_ref[...], sel, preferred_element_type=jnp.float32)
    kap = jnp.clip(jnp.exp(lk_ref[0]), 0.5, 50.0)
    o_ref[...] = jax.nn.sigmoid(kap * (z - th_ref[0]))


def _tc_part(x, th, log_kappa, i_idx, j_idx, fb=2176):
    batch, dim = x.shape
    num_facts = i_idx.shape[0]
    nb = num_facts // fb
    ii3 = i_idx.reshape(nb, 1, fb)
    jj3 = j_idx.reshape(nb, 1, fb)
    th3 = th.reshape(nb, 1, fb)
    lk3 = log_kappa.reshape(nb, 1, fb)
    spec1 = pl.BlockSpec((1, 1, fb), lambda f: (f, 0, 0))
    return pl.pallas_call(
        _tc_kernel_body,
        grid=(nb,),
        in_specs=[spec1, spec1, spec1, spec1,
                  pl.BlockSpec((batch, dim), lambda f: (0, 0))],
        out_specs=pl.BlockSpec((batch, fb), lambda f: (0, f)),
        out_shape=jax.ShapeDtypeStruct((batch, num_facts), jnp.float32),
    )(ii3, jj3, th3, lk3, x)


SC_ROWS = 0


def kernel(x, th, log_kappa, i_idx, j_idx):
    batch = x.shape[0]
    sc_rows = SC_ROWS
    if sc_rows == 0:
        return _tc_part(x, th, log_kappa, i_idx, j_idx)
    if sc_rows >= batch:
        return _sc_part(x, th, log_kappa, i_idx, j_idx)
    tc_out = _tc_part(x[:batch - sc_rows], th, log_kappa, i_idx, j_idx)
    sc_out = _sc_part(x[batch - sc_rows:], th, log_kappa, i_idx, j_idx)
    return jnp.concatenate([tc_out, sc_out], axis=0)
